# baseline (speedup 1.0000x reference)
import functools

import jax
import jax.numpy as jnp
from jax import lax
from jax.experimental import pallas as pl
from jax.experimental.pallas import tpu as pltpu
from jax.experimental.pallas import tpu_sc as plsc

N_NODES = 10000
EMB = 128
N_REL = 4
N_EDGES = 320000
SEG = N_REL * N_NODES

CW = 32
NCHUNK = EMB // CW
GS = 128
NGS = 20
SBS = GS * NGS
RING = 8
SLAG = 4
TILE_EDGES_P = 20480
N_EDGES_P = 16 * TILE_EDGES_P
NSUPER_S = TILE_EDGES_P // SBS
SEG_P = SEG + 72
ROWS_PER_TILE = SEG // 16
ZR = 100

G = 80
NG = 25
SB = G * NG
CNT_TILE_EDGES = N_EDGES // 32
NSUPER_CNT = CNT_TILE_EDGES // SB
BN = 1000


def _sc_body(src_hbm, dst_hbm, et_hbm, x2d_hbm, acc_out,
             acc_s, srcv, dstv, etv, gidx, segb, rows, zbuf, gsem, ssem):
    c = lax.axis_index("c")
    s = lax.axis_index("s")
    tile_base = s * TILE_EDGES_P
    acc_base = s * ROWS_PER_TILE

    def zfill(i, carry):
        zbuf[i, pl.ds(0, 16)] = jnp.zeros((16,), jnp.float32)
        zbuf[i, pl.ds(16, 16)] = jnp.zeros((16,), jnp.float32)
        return carry
    lax.fori_loop(0, ZR, zfill, 0)

    for p in range(2):
        chunk = c * 2 + p

        def zero_body(i, carry):
            pltpu.sync_copy(zbuf, acc_s.at[pl.ds(acc_base + i * ZR, ZR)])
            return carry
        lax.fori_loop(0, ROWS_PER_TILE // ZR, zero_body, 0)
        plsc.subcore_barrier()

        def super_body(sb, carry):
            base = tile_base + sb * SBS
            pltpu.make_async_copy(
                src_hbm.at[pl.ds(base, SBS)], srcv, gsem).start()
            pltpu.make_async_copy(
                dst_hbm.at[pl.ds(base, SBS)], dstv, gsem).start()
            pltpu.make_async_copy(
                et_hbm.at[pl.ds(base, SBS)], etv, gsem).start()
            pltpu.make_async_copy(
                src_hbm.at[pl.ds(base, SBS)], srcv, gsem).wait()
            pltpu.make_async_copy(
                dst_hbm.at[pl.ds(base, SBS)], dstv, gsem).wait()
            pltpu.make_async_copy(
                et_hbm.at[pl.ds(base, SBS)], etv, gsem).wait()

            def idx_body(g, carry2):
                o = g * GS
                for k in range(GS // 16):
                    sv = srcv[pl.ds(o + 16 * k, 16)]
                    dv = dstv[pl.ds(o + 16 * k, 16)]
                    ev = etv[pl.ds(o + 16 * k, 16)]
                    gidx[g, pl.ds(16 * k, 16)] = sv * NCHUNK + chunk
                    segb[g, pl.ds(16 * k, 16)] = dv * N_REL + ev
                return carry2
            lax.fori_loop(0, NGS, idx_body, 0)

            def fire(g, carry2):
                pltpu.make_async_copy(
                    x2d_hbm.at[gidx.at[g]], rows.at[g], gsem).start()
                return carry2
            lax.fori_loop(0, min(RING, NGS), fire, 0)

            def pipe_body(g, carry2):
                slot = lax.rem(g, RING)
                pltpu.make_async_copy(
                    x2d_hbm.at[gidx.at[g]], rows.at[slot], gsem).wait()
                pltpu.async_copy(rows.at[slot], acc_s.at[segb.at[g]],
                                 ssem, add=True)

                @pl.when(g >= SLAG)
                def _():
                    h = g - SLAG
                    hslot = lax.rem(h, RING)
                    pltpu.make_async_copy(
                        rows.at[hslot], acc_s.at[segb.at[h]], ssem).wait()

                    @pl.when(h + RING < NGS)
                    def _():
                        nxt = h + RING
                        pltpu.make_async_copy(
                            x2d_hbm.at[gidx.at[nxt]],
                            rows.at[lax.rem(nxt, RING)], gsem).start()
                return carry2
            lax.fori_loop(0, NGS, pipe_body, 0)

            def retire(t, carry2):
                h = NGS - SLAG + t
                pltpu.make_async_copy(
                    rows.at[lax.rem(h, RING)], acc_s.at[segb.at[h]],
                    ssem).wait()
                return carry2
            lax.fori_loop(0, SLAG, retire, 0)
            return carry
        lax.fori_loop(0, NSUPER_S, super_body, 0)

        plsc.subcore_barrier()
        pltpu.sync_copy(
            acc_s.at[pl.ds(acc_base, ROWS_PER_TILE)],
            acc_out.at[chunk, pl.ds(acc_base, ROWS_PER_TILE)])
        plsc.subcore_barrier()


def _make_sc_segsum():
    mesh = plsc.VectorSubcoreMesh(core_axis_name="c", subcore_axis_name="s")
    scratch = (
        pltpu.VMEM_SHARED((SEG_P, CW), jnp.float32),
        pltpu.VMEM((SBS,), jnp.int32),
        pltpu.VMEM((SBS,), jnp.int32),
        pltpu.VMEM((SBS,), jnp.int32),
        pltpu.VMEM((NGS, GS), jnp.int32),
        pltpu.VMEM((NGS, GS), jnp.int32),
        pltpu.VMEM((RING, GS, CW), jnp.float32),
        pltpu.VMEM((ZR, CW), jnp.float32),
        pltpu.SemaphoreType.DMA,
        pltpu.SemaphoreType.DMA,
    )
    return pl.kernel(
        _sc_body, mesh=mesh,
        out_type=(jax.ShapeDtypeStruct((NCHUNK, SEG, CW), jnp.float32),),
        scratch_types=scratch,
        compiler_params=pltpu.CompilerParams(use_tc_tiling_on_sc=False))


def _cnt_body(dst_hbm, et_hbm, cnt_out,
              cnt_s, dstv, etv, segb, ones, zbuf16):
    c = lax.axis_index("c")
    s = lax.axis_index("s")
    tile_base = (c * 16 + s) * CNT_TILE_EDGES
    acc_base = s * ROWS_PER_TILE

    def zfill(i, carry):
        zbuf16[i, pl.ds(0, 16)] = jnp.zeros((16,), jnp.float32)
        return carry
    lax.fori_loop(0, ZR, zfill, 0)

    def ofill(i, carry):
        ones[i, pl.ds(0, 16)] = jnp.ones((16,), jnp.float32)
        return carry
    lax.fori_loop(0, G, ofill, 0)

    def zero_body(i, carry):
        pltpu.sync_copy(zbuf16, cnt_s.at[pl.ds(acc_base + i * ZR, ZR)])
        return carry
    lax.fori_loop(0, ROWS_PER_TILE // ZR, zero_body, 0)
    plsc.subcore_barrier()

    def super_body(sb, carry):
        base = tile_base + sb * SB
        pltpu.sync_copy(dst_hbm.at[pl.ds(base, SB)], dstv)
        pltpu.sync_copy(et_hbm.at[pl.ds(base, SB)], etv)

        def idx_body(g, carry2):
            o = g * G
            for k in range(G // 16):
                dv = dstv[pl.ds(o + 16 * k, 16)]
                ev = etv[pl.ds(o + 16 * k, 16)]
                segb[g, pl.ds(16 * k, 16)] = dv * N_REL + ev
            return carry2
        lax.fori_loop(0, NG, idx_body, 0)

        def scat_body(g, carry2):
            pltpu.sync_copy(ones, cnt_s.at[segb.at[g]], add=True)
            return carry2
        lax.fori_loop(0, NG, scat_body, 0)
        return carry
    lax.fori_loop(0, NSUPER_CNT, super_body, 0)
    plsc.subcore_barrier()

    pltpu.sync_copy(cnt_s.at[pl.ds(acc_base, ROWS_PER_TILE)],
                    cnt_out.at[c, pl.ds(acc_base, ROWS_PER_TILE)])


def _make_sc_counts():
    mesh = plsc.VectorSubcoreMesh(core_axis_name="c", subcore_axis_name="s")
    scratch = (
        pltpu.VMEM_SHARED((SEG, 16), jnp.float32),
        pltpu.VMEM((SB,), jnp.int32),
        pltpu.VMEM((SB,), jnp.int32),
        pltpu.VMEM((NG, G), jnp.int32),
        pltpu.VMEM((G, 16), jnp.float32),
        pltpu.VMEM((ZR, 16), jnp.float32),
    )
    return pl.kernel(
        _cnt_body, mesh=mesh,
        out_type=(jax.ShapeDtypeStruct((2, SEG, 16), jnp.float32),),
        scratch_types=scratch,
        compiler_params=pltpu.CompilerParams(use_tc_tiling_on_sc=False))


def _dense_body(final_avg, x_ref, acc_ref, inv_ref, root_ref, w_ref, b_ref,
                x0_ref, out_ref):
    x = x_ref[...]
    out = jnp.dot(x, root_ref[...], preferred_element_type=jnp.float32)
    out += b_ref[...]
    inv = inv_ref[...]
    scaled = [acc_ref[ch] * inv for ch in range(NCHUNK)]
    pieces = [scaled[ch][:, r * CW:(r + 1) * CW]
              for r in range(N_REL) for ch in range(NCHUNK)]
    bcat = jnp.concatenate(pieces, axis=1)
    out += jnp.dot(bcat, w_ref[...], preferred_element_type=jnp.float32)
    if final_avg:
        out_ref[...] = (x0_ref[...] + x + out) * (1.0 / 3.0)
    else:
        out_ref[...] = out


def _dense_layer(x, acc_t, inv_t, root, wstack, b, x0, final_avg):
    grid = (N_NODES // BN,)
    return pl.pallas_call(
        functools.partial(_dense_body, final_avg),
        grid=grid,
        in_specs=[
            pl.BlockSpec((BN, EMB), lambda i: (i, 0)),
            pl.BlockSpec((NCHUNK, BN, EMB), lambda i: (0, i, 0)),
            pl.BlockSpec((BN, EMB), lambda i: (i, 0)),
            pl.BlockSpec((EMB, EMB), lambda i: (0, 0)),
            pl.BlockSpec((N_REL * EMB, EMB), lambda i: (0, 0)),
            pl.BlockSpec((1, EMB), lambda i: (0, 0)),
            pl.BlockSpec((BN, EMB), lambda i: (i, 0)),
        ],
        out_specs=pl.BlockSpec((BN, EMB), lambda i: (i, 0)),
        out_shape=jax.ShapeDtypeStruct((N_NODES, EMB), jnp.float32),
    )(x, acc_t, inv_t, root, wstack, b, x0)


_sc_segsum = _make_sc_segsum()
_sc_counts = _make_sc_counts()


def kernel(edge_index_mp, edge_type, emb, w0, root0, b0, w1, root1, b1):
    src = edge_index_mp[0]
    dst = edge_index_mp[1]

    npad = N_EDGES_P - N_EDGES
    pad_iota = jax.lax.iota(jnp.int32, npad)
    src_p = jnp.concatenate([src, jnp.zeros((npad,), jnp.int32)])
    dst_p = jnp.concatenate([dst, N_NODES + (pad_iota % 16)])
    et_p = jnp.concatenate([edge_type, jnp.zeros((npad,), jnp.int32)])

    (cnt16,) = _sc_counts(dst, edge_type)
    cnts = cnt16[0, :, 0] + cnt16[1, :, 0]
    inv = 1.0 / jnp.clip(cnts.reshape(N_NODES, N_REL), 1.0, None)
    inv_t = jnp.repeat(inv, CW, axis=1)

    (acc0,) = _sc_segsum(src_p, dst_p, et_p, emb.reshape(SEG, CW))
    x1 = _dense_layer(emb, acc0.reshape(NCHUNK, N_NODES, EMB), inv_t,
                      root0, w0.reshape(N_REL * EMB, EMB),
                      b0.reshape(1, EMB), emb, False)
    (acc1,) = _sc_segsum(src_p, dst_p, et_p, x1.reshape(SEG, CW))
    x2f = _dense_layer(x1, acc1.reshape(NCHUNK, N_NODES, EMB), inv_t,
                       root1, w1.reshape(N_REL * EMB, EMB),
                       b1.reshape(1, EMB), emb, True)
    return x2f


# speedup vs baseline: 2.4177x; 2.4177x over previous
import functools

import jax
import jax.numpy as jnp
from jax import lax
from jax.experimental import pallas as pl
from jax.experimental.pallas import tpu as pltpu
from jax.experimental.pallas import tpu_sc as plsc

N_NODES = 10000
EMB = 128
N_REL = 4
N_EDGES = 320000
SEG = N_REL * N_NODES

CW = 32
NCHUNK = EMB // CW
GS = 128
NGS = 20
SBS = GS * NGS
RING = 8
SLAG = 4
TILE_EDGES_P = 20480
N_EDGES_P = 16 * TILE_EDGES_P
NSUPER_S = TILE_EDGES_P // SBS
SEG_P = SEG + 72
ROWS_PER_TILE = SEG // 16
ZR = 100

G = 80
NG = 25
SB = G * NG
CNT_TILE_EDGES = N_EDGES // 32
NSUPER_CNT = CNT_TILE_EDGES // SB
BN = 1000


def _sc_body(src_hbm, dst_hbm, et_hbm, x2d_hbm, acc_out,
             acc_s, srcv, dstv, etv, gidx, segb, rows, zbuf, gsem, ssem):
    c = lax.axis_index("c")
    s = lax.axis_index("s")
    tile_base = s * TILE_EDGES_P
    acc_base = s * ROWS_PER_TILE

    def zfill(i, carry):
        zbuf[i, pl.ds(0, 16)] = jnp.zeros((16,), jnp.float32)
        zbuf[i, pl.ds(16, 16)] = jnp.zeros((16,), jnp.float32)
        return carry
    lax.fori_loop(0, ZR, zfill, 0)

    for p in range(2):
        chunk = c * 2 + p

        def zero_body(i, carry):
            pltpu.sync_copy(zbuf, acc_s.at[pl.ds(acc_base + i * ZR, ZR)])
            return carry
        lax.fori_loop(0, ROWS_PER_TILE // ZR, zero_body, 0)
        plsc.subcore_barrier()

        def super_body(sb, carry):
            base = tile_base + sb * SBS
            pltpu.make_async_copy(
                src_hbm.at[pl.ds(base, SBS)], srcv, gsem).start()
            pltpu.make_async_copy(
                dst_hbm.at[pl.ds(base, SBS)], dstv, gsem).start()
            pltpu.make_async_copy(
                et_hbm.at[pl.ds(base, SBS)], etv, gsem).start()
            pltpu.make_async_copy(
                src_hbm.at[pl.ds(base, SBS)], srcv, gsem).wait()
            pltpu.make_async_copy(
                dst_hbm.at[pl.ds(base, SBS)], dstv, gsem).wait()
            pltpu.make_async_copy(
                et_hbm.at[pl.ds(base, SBS)], etv, gsem).wait()

            def idx_body(g, carry2):
                o = g * GS
                for k in range(GS // 16):
                    sv = srcv[pl.ds(o + 16 * k, 16)]
                    dv = dstv[pl.ds(o + 16 * k, 16)]
                    ev = etv[pl.ds(o + 16 * k, 16)]
                    gidx[g, pl.ds(16 * k, 16)] = sv * NCHUNK + chunk
                    segb[g, pl.ds(16 * k, 16)] = dv * N_REL + ev
                return carry2
            lax.fori_loop(0, NGS, idx_body, 0)

            def fire(g, carry2):
                pltpu.make_async_copy(
                    x2d_hbm.at[gidx.at[g]], rows.at[g], gsem).start()
                return carry2
            lax.fori_loop(0, min(RING, NGS), fire, 0)

            def pipe_body(g, carry2):
                slot = lax.rem(g, RING)
                pltpu.make_async_copy(
                    x2d_hbm.at[gidx.at[g]], rows.at[slot], gsem).wait()
                pltpu.async_copy(rows.at[slot], acc_s.at[segb.at[g]],
                                 ssem, add=True)

                @pl.when(g >= SLAG)
                def _():
                    h = g - SLAG
                    hslot = lax.rem(h, RING)
                    pltpu.make_async_copy(
                        rows.at[hslot], acc_s.at[segb.at[h]], ssem).wait()

                    @pl.when(h + RING < NGS)
                    def _():
                        nxt = h + RING
                        pltpu.make_async_copy(
                            x2d_hbm.at[gidx.at[nxt]],
                            rows.at[lax.rem(nxt, RING)], gsem).start()
                return carry2
            lax.fori_loop(0, NGS, pipe_body, 0)

            def retire(t, carry2):
                h = NGS - SLAG + t
                pltpu.make_async_copy(
                    rows.at[lax.rem(h, RING)], acc_s.at[segb.at[h]],
                    ssem).wait()
                return carry2
            lax.fori_loop(0, SLAG, retire, 0)
            return carry
        lax.fori_loop(0, NSUPER_S, super_body, 0)

        plsc.subcore_barrier()
        pltpu.sync_copy(
            acc_s.at[pl.ds(acc_base, ROWS_PER_TILE)],
            acc_out.at[chunk, pl.ds(acc_base, ROWS_PER_TILE)])
        plsc.subcore_barrier()


def _make_sc_segsum():
    mesh = plsc.VectorSubcoreMesh(core_axis_name="c", subcore_axis_name="s")
    scratch = (
        pltpu.VMEM_SHARED((SEG_P, CW), jnp.float32),
        pltpu.VMEM((SBS,), jnp.int32),
        pltpu.VMEM((SBS,), jnp.int32),
        pltpu.VMEM((SBS,), jnp.int32),
        pltpu.VMEM((NGS, GS), jnp.int32),
        pltpu.VMEM((NGS, GS), jnp.int32),
        pltpu.VMEM((RING, GS, CW), jnp.float32),
        pltpu.VMEM((ZR, CW), jnp.float32),
        pltpu.SemaphoreType.DMA,
        pltpu.SemaphoreType.DMA,
    )
    return pl.kernel(
        _sc_body, mesh=mesh,
        out_type=(jax.ShapeDtypeStruct((NCHUNK, SEG, CW), jnp.float32),),
        scratch_types=scratch,
        compiler_params=pltpu.CompilerParams(use_tc_tiling_on_sc=False))


def _cnt_body(dst_hbm, et_hbm, cnt_out,
              cnt_s, dstv, etv, segb, ones, zbuf16):
    c = lax.axis_index("c")
    s = lax.axis_index("s")
    tile_base = (c * 16 + s) * CNT_TILE_EDGES
    acc_base = s * ROWS_PER_TILE

    def zfill(i, carry):
        zbuf16[i, pl.ds(0, 16)] = jnp.zeros((16,), jnp.float32)
        return carry
    lax.fori_loop(0, ZR, zfill, 0)

    def ofill(i, carry):
        ones[i, pl.ds(0, 16)] = jnp.ones((16,), jnp.float32)
        return carry
    lax.fori_loop(0, G, ofill, 0)

    def zero_body(i, carry):
        pltpu.sync_copy(zbuf16, cnt_s.at[pl.ds(acc_base + i * ZR, ZR)])
        return carry
    lax.fori_loop(0, ROWS_PER_TILE // ZR, zero_body, 0)
    plsc.subcore_barrier()

    def super_body(sb, carry):
        base = tile_base + sb * SB
        pltpu.sync_copy(dst_hbm.at[pl.ds(base, SB)], dstv)
        pltpu.sync_copy(et_hbm.at[pl.ds(base, SB)], etv)

        def idx_body(g, carry2):
            o = g * G
            for k in range(G // 16):
                dv = dstv[pl.ds(o + 16 * k, 16)]
                ev = etv[pl.ds(o + 16 * k, 16)]
                segb[g, pl.ds(16 * k, 16)] = dv * N_REL + ev
            return carry2
        lax.fori_loop(0, NG, idx_body, 0)

        def scat_body(g, carry2):
            pltpu.sync_copy(ones, cnt_s.at[segb.at[g]], add=True)
            return carry2
        lax.fori_loop(0, NG, scat_body, 0)
        return carry
    lax.fori_loop(0, NSUPER_CNT, super_body, 0)
    plsc.subcore_barrier()

    pltpu.sync_copy(cnt_s.at[pl.ds(acc_base, ROWS_PER_TILE)],
                    cnt_out.at[c, pl.ds(acc_base, ROWS_PER_TILE)])


def _make_sc_counts():
    mesh = plsc.VectorSubcoreMesh(core_axis_name="c", subcore_axis_name="s")
    scratch = (
        pltpu.VMEM_SHARED((SEG, 16), jnp.float32),
        pltpu.VMEM((SB,), jnp.int32),
        pltpu.VMEM((SB,), jnp.int32),
        pltpu.VMEM((NG, G), jnp.int32),
        pltpu.VMEM((G, 16), jnp.float32),
        pltpu.VMEM((ZR, 16), jnp.float32),
    )
    return pl.kernel(
        _cnt_body, mesh=mesh,
        out_type=(jax.ShapeDtypeStruct((2, SEG, 16), jnp.float32),),
        scratch_types=scratch,
        compiler_params=pltpu.CompilerParams(use_tc_tiling_on_sc=False))


def _dense_body(final_avg, x_ref, acc_ref, inv_ref, root_ref, w_ref, b_ref,
                x0_ref, out_ref):
    x = x_ref[...]
    out = jnp.dot(x, root_ref[...], preferred_element_type=jnp.float32)
    out += b_ref[...]
    inv = inv_ref[...]
    scaled = [acc_ref[ch] * inv for ch in range(NCHUNK)]
    pieces = [scaled[ch][:, r * CW:(r + 1) * CW]
              for r in range(N_REL) for ch in range(NCHUNK)]
    bcat = jnp.concatenate(pieces, axis=1)
    out += jnp.dot(bcat, w_ref[...], preferred_element_type=jnp.float32)
    if final_avg:
        out_ref[...] = (x0_ref[...] + x + out) * (1.0 / 3.0)
    else:
        out_ref[...] = out


def _dense_layer(x, acc_t, inv_t, root, wstack, b, x0, final_avg):
    grid = (N_NODES // BN,)
    return pl.pallas_call(
        functools.partial(_dense_body, final_avg),
        grid=grid,
        in_specs=[
            pl.BlockSpec((BN, EMB), lambda i: (i, 0)),
            pl.BlockSpec((NCHUNK, BN, EMB), lambda i: (0, i, 0)),
            pl.BlockSpec((BN, EMB), lambda i: (i, 0)),
            pl.BlockSpec((EMB, EMB), lambda i: (0, 0)),
            pl.BlockSpec((N_REL * EMB, EMB), lambda i: (0, 0)),
            pl.BlockSpec((1, EMB), lambda i: (0, 0)),
            pl.BlockSpec((BN, EMB), lambda i: (i, 0)),
        ],
        out_specs=pl.BlockSpec((BN, EMB), lambda i: (i, 0)),
        out_shape=jax.ShapeDtypeStruct((N_NODES, EMB), jnp.float32),
    )(x, acc_t, inv_t, root, wstack, b, x0)


_sc_segsum = _make_sc_segsum()
_sc_counts = _make_sc_counts()


def kernel(edge_index_mp, edge_type, emb, w0, root0, b0, w1, root1, b1):
    src = edge_index_mp[0]
    dst = edge_index_mp[1]

    npad = N_EDGES_P - N_EDGES
    pad_iota = jax.lax.iota(jnp.int32, npad)
    src_p = jnp.concatenate([src, pad_iota % N_NODES])
    dst_p = jnp.concatenate([dst, N_NODES + (pad_iota % 16)])
    et_p = jnp.concatenate([edge_type, jnp.zeros((npad,), jnp.int32)])

    (cnt16,) = _sc_counts(dst, edge_type)
    cnts = cnt16[0, :, 0] + cnt16[1, :, 0]
    inv = 1.0 / jnp.clip(cnts.reshape(N_NODES, N_REL), 1.0, None)
    inv_t = jnp.repeat(inv, CW, axis=1)

    (acc0,) = _sc_segsum(src_p, dst_p, et_p, emb.reshape(SEG, CW))
    x1 = _dense_layer(emb, acc0.reshape(NCHUNK, N_NODES, EMB), inv_t,
                      root0, w0.reshape(N_REL * EMB, EMB),
                      b0.reshape(1, EMB), emb, False)
    (acc1,) = _sc_segsum(src_p, dst_p, et_p, x1.reshape(SEG, CW))
    x2f = _dense_layer(x1, acc1.reshape(NCHUNK, N_NODES, EMB), inv_t,
                       root1, w1.reshape(N_REL * EMB, EMB),
                       b1.reshape(1, EMB), emb, True)
    return x2f


# speedup vs baseline: 2.4599x; 1.0175x over previous
import functools

import jax
import jax.numpy as jnp
from jax import lax
from jax.experimental import pallas as pl
from jax.experimental.pallas import tpu as pltpu
from jax.experimental.pallas import tpu_sc as plsc

N_NODES = 10000
EMB = 128
N_REL = 4
N_EDGES = 320000
SEG = N_REL * N_NODES

CW = 32
NCHUNK = EMB // CW
GS = 80
NGS = 25
SBS = GS * NGS
RING = 14
SLAG = 6
TILE_EDGES_P = N_EDGES // 16
NSUPER_S = TILE_EDGES_P // SBS
SEG_P = SEG
ROWS_PER_TILE = SEG // 16
ZR = 100

G = 80
NG = 25
SB = G * NG
CNT_TILE_EDGES = N_EDGES // 32
NSUPER_CNT = CNT_TILE_EDGES // SB
BN = 1000


def _sc_body(src_hbm, dst_hbm, et_hbm, x2d_hbm, acc_out,
             acc_s, srcv, dstv, etv, gidx, segb, rows, zbuf, gsem, ssem):
    c = lax.axis_index("c")
    s = lax.axis_index("s")
    tile_base = s * TILE_EDGES_P
    acc_base = s * ROWS_PER_TILE

    def zfill(i, carry):
        zbuf[i, pl.ds(0, 16)] = jnp.zeros((16,), jnp.float32)
        zbuf[i, pl.ds(16, 16)] = jnp.zeros((16,), jnp.float32)
        return carry
    lax.fori_loop(0, ZR, zfill, 0)

    for p in range(2):
        chunk = c * 2 + p

        def zero_body(i, carry):
            pltpu.sync_copy(zbuf, acc_s.at[pl.ds(acc_base + i * ZR, ZR)])
            return carry
        lax.fori_loop(0, ROWS_PER_TILE // ZR, zero_body, 0)
        plsc.subcore_barrier()

        def super_body(sb, carry):
            base = tile_base + sb * SBS
            pltpu.make_async_copy(
                src_hbm.at[pl.ds(base, SBS)], srcv, gsem).start()
            pltpu.make_async_copy(
                dst_hbm.at[pl.ds(base, SBS)], dstv, gsem).start()
            pltpu.make_async_copy(
                et_hbm.at[pl.ds(base, SBS)], etv, gsem).start()
            pltpu.make_async_copy(
                src_hbm.at[pl.ds(base, SBS)], srcv, gsem).wait()
            pltpu.make_async_copy(
                dst_hbm.at[pl.ds(base, SBS)], dstv, gsem).wait()
            pltpu.make_async_copy(
                et_hbm.at[pl.ds(base, SBS)], etv, gsem).wait()

            def idx_body(g, carry2):
                o = g * GS
                for k in range(GS // 16):
                    sv = srcv[pl.ds(o + 16 * k, 16)]
                    dv = dstv[pl.ds(o + 16 * k, 16)]
                    ev = etv[pl.ds(o + 16 * k, 16)]
                    gidx[g, pl.ds(16 * k, 16)] = sv * NCHUNK + chunk
                    segb[g, pl.ds(16 * k, 16)] = dv * N_REL + ev
                return carry2
            lax.fori_loop(0, NGS, idx_body, 0)

            def fire(g, carry2):
                pltpu.make_async_copy(
                    x2d_hbm.at[gidx.at[g]], rows.at[g], gsem).start()
                return carry2
            lax.fori_loop(0, min(RING, NGS), fire, 0)

            def pipe_body(g, carry2):
                slot = lax.rem(g, RING)
                pltpu.make_async_copy(
                    x2d_hbm.at[gidx.at[g]], rows.at[slot], gsem).wait()
                pltpu.async_copy(rows.at[slot], acc_s.at[segb.at[g]],
                                 ssem, add=True)

                @pl.when(g >= SLAG)
                def _():
                    h = g - SLAG
                    hslot = lax.rem(h, RING)
                    pltpu.make_async_copy(
                        rows.at[hslot], acc_s.at[segb.at[h]], ssem).wait()

                    @pl.when(h + RING < NGS)
                    def _():
                        nxt = h + RING
                        pltpu.make_async_copy(
                            x2d_hbm.at[gidx.at[nxt]],
                            rows.at[lax.rem(nxt, RING)], gsem).start()
                return carry2
            lax.fori_loop(0, NGS, pipe_body, 0)

            def retire(t, carry2):
                h = NGS - SLAG + t
                pltpu.make_async_copy(
                    rows.at[lax.rem(h, RING)], acc_s.at[segb.at[h]],
                    ssem).wait()
                return carry2
            lax.fori_loop(0, SLAG, retire, 0)
            return carry
        lax.fori_loop(0, NSUPER_S, super_body, 0)

        plsc.subcore_barrier()
        pltpu.sync_copy(
            acc_s.at[pl.ds(acc_base, ROWS_PER_TILE)],
            acc_out.at[chunk, pl.ds(acc_base, ROWS_PER_TILE)])
        plsc.subcore_barrier()


def _make_sc_segsum():
    mesh = plsc.VectorSubcoreMesh(core_axis_name="c", subcore_axis_name="s")
    scratch = (
        pltpu.VMEM_SHARED((SEG_P, CW), jnp.float32),
        pltpu.VMEM((SBS,), jnp.int32),
        pltpu.VMEM((SBS,), jnp.int32),
        pltpu.VMEM((SBS,), jnp.int32),
        pltpu.VMEM((NGS, GS), jnp.int32),
        pltpu.VMEM((NGS, GS), jnp.int32),
        pltpu.VMEM((RING, GS, CW), jnp.float32),
        pltpu.VMEM((ZR, CW), jnp.float32),
        pltpu.SemaphoreType.DMA,
        pltpu.SemaphoreType.DMA,
    )
    return pl.kernel(
        _sc_body, mesh=mesh,
        out_type=(jax.ShapeDtypeStruct((NCHUNK, SEG, CW), jnp.float32),),
        scratch_types=scratch,
        compiler_params=pltpu.CompilerParams(use_tc_tiling_on_sc=False))


def _cnt_body(dst_hbm, et_hbm, cnt_out,
              cnt_s, dstv, etv, segb, ones, zbuf16):
    c = lax.axis_index("c")
    s = lax.axis_index("s")
    tile_base = (c * 16 + s) * CNT_TILE_EDGES
    acc_base = s * ROWS_PER_TILE

    def zfill(i, carry):
        zbuf16[i, pl.ds(0, 16)] = jnp.zeros((16,), jnp.float32)
        return carry
    lax.fori_loop(0, ZR, zfill, 0)

    def ofill(i, carry):
        ones[i, pl.ds(0, 16)] = jnp.ones((16,), jnp.float32)
        return carry
    lax.fori_loop(0, G, ofill, 0)

    def zero_body(i, carry):
        pltpu.sync_copy(zbuf16, cnt_s.at[pl.ds(acc_base + i * ZR, ZR)])
        return carry
    lax.fori_loop(0, ROWS_PER_TILE // ZR, zero_body, 0)
    plsc.subcore_barrier()

    def super_body(sb, carry):
        base = tile_base + sb * SB
        pltpu.sync_copy(dst_hbm.at[pl.ds(base, SB)], dstv)
        pltpu.sync_copy(et_hbm.at[pl.ds(base, SB)], etv)

        def idx_body(g, carry2):
            o = g * G
            for k in range(G // 16):
                dv = dstv[pl.ds(o + 16 * k, 16)]
                ev = etv[pl.ds(o + 16 * k, 16)]
                segb[g, pl.ds(16 * k, 16)] = dv * N_REL + ev
            return carry2
        lax.fori_loop(0, NG, idx_body, 0)

        def scat_body(g, carry2):
            pltpu.sync_copy(ones, cnt_s.at[segb.at[g]], add=True)
            return carry2
        lax.fori_loop(0, NG, scat_body, 0)
        return carry
    lax.fori_loop(0, NSUPER_CNT, super_body, 0)
    plsc.subcore_barrier()

    pltpu.sync_copy(cnt_s.at[pl.ds(acc_base, ROWS_PER_TILE)],
                    cnt_out.at[c, pl.ds(acc_base, ROWS_PER_TILE)])


def _make_sc_counts():
    mesh = plsc.VectorSubcoreMesh(core_axis_name="c", subcore_axis_name="s")
    scratch = (
        pltpu.VMEM_SHARED((SEG, 16), jnp.float32),
        pltpu.VMEM((SB,), jnp.int32),
        pltpu.VMEM((SB,), jnp.int32),
        pltpu.VMEM((NG, G), jnp.int32),
        pltpu.VMEM((G, 16), jnp.float32),
        pltpu.VMEM((ZR, 16), jnp.float32),
    )
    return pl.kernel(
        _cnt_body, mesh=mesh,
        out_type=(jax.ShapeDtypeStruct((2, SEG, 16), jnp.float32),),
        scratch_types=scratch,
        compiler_params=pltpu.CompilerParams(use_tc_tiling_on_sc=False))


def _dense_body(final_avg, x_ref, acc_ref, inv_ref, root_ref, w_ref, b_ref,
                x0_ref, out_ref):
    x = x_ref[...]
    out = jnp.dot(x, root_ref[...], preferred_element_type=jnp.float32)
    out += b_ref[...]
    inv = inv_ref[...]
    scaled = [acc_ref[ch] * inv for ch in range(NCHUNK)]
    pieces = [scaled[ch][:, r * CW:(r + 1) * CW]
              for r in range(N_REL) for ch in range(NCHUNK)]
    bcat = jnp.concatenate(pieces, axis=1)
    out += jnp.dot(bcat, w_ref[...], preferred_element_type=jnp.float32)
    if final_avg:
        out_ref[...] = (x0_ref[...] + x + out) * (1.0 / 3.0)
    else:
        out_ref[...] = out


def _dense_layer(x, acc_t, inv_t, root, wstack, b, x0, final_avg):
    grid = (N_NODES // BN,)
    return pl.pallas_call(
        functools.partial(_dense_body, final_avg),
        grid=grid,
        in_specs=[
            pl.BlockSpec((BN, EMB), lambda i: (i, 0)),
            pl.BlockSpec((NCHUNK, BN, EMB), lambda i: (0, i, 0)),
            pl.BlockSpec((BN, EMB), lambda i: (i, 0)),
            pl.BlockSpec((EMB, EMB), lambda i: (0, 0)),
            pl.BlockSpec((N_REL * EMB, EMB), lambda i: (0, 0)),
            pl.BlockSpec((1, EMB), lambda i: (0, 0)),
            pl.BlockSpec((BN, EMB), lambda i: (i, 0)),
        ],
        out_specs=pl.BlockSpec((BN, EMB), lambda i: (i, 0)),
        out_shape=jax.ShapeDtypeStruct((N_NODES, EMB), jnp.float32),
    )(x, acc_t, inv_t, root, wstack, b, x0)


_sc_segsum = _make_sc_segsum()
_sc_counts = _make_sc_counts()


def kernel(edge_index_mp, edge_type, emb, w0, root0, b0, w1, root1, b1):
    src = edge_index_mp[0]
    dst = edge_index_mp[1]

    (cnt16,) = _sc_counts(dst, edge_type)
    cnts = cnt16[0, :, 0] + cnt16[1, :, 0]
    inv = 1.0 / jnp.clip(cnts.reshape(N_NODES, N_REL), 1.0, None)
    inv_t = jnp.repeat(inv, CW, axis=1)

    (acc0,) = _sc_segsum(src, dst, edge_type, emb.reshape(SEG, CW))
    x1 = _dense_layer(emb, acc0.reshape(NCHUNK, N_NODES, EMB), inv_t,
                      root0, w0.reshape(N_REL * EMB, EMB),
                      b0.reshape(1, EMB), emb, False)
    (acc1,) = _sc_segsum(src, dst, edge_type, x1.reshape(SEG, CW))
    x2f = _dense_layer(x1, acc1.reshape(NCHUNK, N_NODES, EMB), inv_t,
                       root1, w1.reshape(N_REL * EMB, EMB),
                       b1.reshape(1, EMB), emb, True)
    return x2f


# speedup vs baseline: 2.4996x; 1.0161x over previous
import functools

import jax
import jax.numpy as jnp
from jax import lax
from jax.experimental import pallas as pl
from jax.experimental.pallas import tpu as pltpu
from jax.experimental.pallas import tpu_sc as plsc

N_NODES = 10000
EMB = 128
N_REL = 4
N_EDGES = 320000
SEG = N_REL * N_NODES

CW = 32
NCHUNK = EMB // CW
GS = 80
NGS = 25
SBS = GS * NGS
RING = 14
SLAG = 6
TILE_EDGES_P = N_EDGES // 16
NSUPER_S = TILE_EDGES_P // SBS
SEG_P = SEG
ROWS_PER_TILE = SEG // 16
ZR = 100

G = 80
NG = 25
SB = G * NG
CNT_TILE_EDGES = N_EDGES // 32
NSUPER_CNT = CNT_TILE_EDGES // SB
BN = 2000


def _sc_body(src_hbm, dst_hbm, et_hbm, x2d_hbm, acc_out,
             acc_s, srcv, dstv, etv, gidx, segb, rows, zbuf, gsem, ssem):
    c = lax.axis_index("c")
    s = lax.axis_index("s")
    tile_base = s * TILE_EDGES_P
    acc_base = s * ROWS_PER_TILE

    def zfill(i, carry):
        zbuf[i, pl.ds(0, 16)] = jnp.zeros((16,), jnp.float32)
        zbuf[i, pl.ds(16, 16)] = jnp.zeros((16,), jnp.float32)
        return carry
    lax.fori_loop(0, ZR, zfill, 0)

    for p in range(2):
        chunk = c * 2 + p

        def zero_body(i, carry):
            pltpu.make_async_copy(
                zbuf, acc_s.at[pl.ds(acc_base + i * ZR, ZR)], ssem).start()
            return carry
        lax.fori_loop(0, ROWS_PER_TILE // ZR, zero_body, 0)

        def zero_wait(i, carry):
            pltpu.make_async_copy(
                zbuf, acc_s.at[pl.ds(acc_base + i * ZR, ZR)], ssem).wait()
            return carry
        lax.fori_loop(0, ROWS_PER_TILE // ZR, zero_wait, 0)
        plsc.subcore_barrier()

        def super_body(sb, carry):
            base = tile_base + sb * SBS
            pltpu.make_async_copy(
                src_hbm.at[pl.ds(base, SBS)], srcv, gsem).start()
            pltpu.make_async_copy(
                dst_hbm.at[pl.ds(base, SBS)], dstv, gsem).start()
            pltpu.make_async_copy(
                et_hbm.at[pl.ds(base, SBS)], etv, gsem).start()
            pltpu.make_async_copy(
                src_hbm.at[pl.ds(base, SBS)], srcv, gsem).wait()
            pltpu.make_async_copy(
                dst_hbm.at[pl.ds(base, SBS)], dstv, gsem).wait()
            pltpu.make_async_copy(
                et_hbm.at[pl.ds(base, SBS)], etv, gsem).wait()

            def idx_body(g, carry2):
                o = g * GS
                for k in range(GS // 16):
                    sv = srcv[pl.ds(o + 16 * k, 16)]
                    dv = dstv[pl.ds(o + 16 * k, 16)]
                    ev = etv[pl.ds(o + 16 * k, 16)]
                    gidx[g, pl.ds(16 * k, 16)] = sv * NCHUNK + chunk
                    segb[g, pl.ds(16 * k, 16)] = dv * N_REL + ev
                return carry2
            lax.fori_loop(0, NGS, idx_body, 0)

            def fire(g, carry2):
                pltpu.make_async_copy(
                    x2d_hbm.at[gidx.at[g]], rows.at[g], gsem).start()
                return carry2
            lax.fori_loop(0, min(RING, NGS), fire, 0)

            def pipe_body(g, carry2):
                slot = lax.rem(g, RING)
                pltpu.make_async_copy(
                    x2d_hbm.at[gidx.at[g]], rows.at[slot], gsem).wait()
                pltpu.async_copy(rows.at[slot], acc_s.at[segb.at[g]],
                                 ssem, add=True)

                @pl.when(g >= SLAG)
                def _():
                    h = g - SLAG
                    hslot = lax.rem(h, RING)
                    pltpu.make_async_copy(
                        rows.at[hslot], acc_s.at[segb.at[h]], ssem).wait()

                    @pl.when(h + RING < NGS)
                    def _():
                        nxt = h + RING
                        pltpu.make_async_copy(
                            x2d_hbm.at[gidx.at[nxt]],
                            rows.at[lax.rem(nxt, RING)], gsem).start()
                return carry2
            lax.fori_loop(0, NGS, pipe_body, 0)

            def retire(t, carry2):
                h = NGS - SLAG + t
                pltpu.make_async_copy(
                    rows.at[lax.rem(h, RING)], acc_s.at[segb.at[h]],
                    ssem).wait()
                return carry2
            lax.fori_loop(0, SLAG, retire, 0)
            return carry
        lax.fori_loop(0, NSUPER_S, super_body, 0)

        plsc.subcore_barrier()
        pltpu.sync_copy(
            acc_s.at[pl.ds(acc_base, ROWS_PER_TILE)],
            acc_out.at[chunk, pl.ds(acc_base, ROWS_PER_TILE)])
        plsc.subcore_barrier()


def _make_sc_segsum():
    mesh = plsc.VectorSubcoreMesh(core_axis_name="c", subcore_axis_name="s")
    scratch = (
        pltpu.VMEM_SHARED((SEG_P, CW), jnp.float32),
        pltpu.VMEM((SBS,), jnp.int32),
        pltpu.VMEM((SBS,), jnp.int32),
        pltpu.VMEM((SBS,), jnp.int32),
        pltpu.VMEM((NGS, GS), jnp.int32),
        pltpu.VMEM((NGS, GS), jnp.int32),
        pltpu.VMEM((RING, GS, CW), jnp.float32),
        pltpu.VMEM((ZR, CW), jnp.float32),
        pltpu.SemaphoreType.DMA,
        pltpu.SemaphoreType.DMA,
    )
    return pl.kernel(
        _sc_body, mesh=mesh,
        out_type=(jax.ShapeDtypeStruct((NCHUNK, SEG, CW), jnp.float32),),
        scratch_types=scratch,
        compiler_params=pltpu.CompilerParams(use_tc_tiling_on_sc=False))


def _cnt_body(dst_hbm, et_hbm, cnt_out,
              cnt_s, dstv, etv, segb, ones, zbuf16):
    c = lax.axis_index("c")
    s = lax.axis_index("s")
    tile_base = (c * 16 + s) * CNT_TILE_EDGES
    acc_base = s * ROWS_PER_TILE

    def zfill(i, carry):
        zbuf16[i, pl.ds(0, 16)] = jnp.zeros((16,), jnp.float32)
        return carry
    lax.fori_loop(0, ZR, zfill, 0)

    def ofill(i, carry):
        ones[i, pl.ds(0, 16)] = jnp.ones((16,), jnp.float32)
        return carry
    lax.fori_loop(0, G, ofill, 0)

    def zero_body(i, carry):
        pltpu.sync_copy(zbuf16, cnt_s.at[pl.ds(acc_base + i * ZR, ZR)])
        return carry
    lax.fori_loop(0, ROWS_PER_TILE // ZR, zero_body, 0)
    plsc.subcore_barrier()

    def super_body(sb, carry):
        base = tile_base + sb * SB
        pltpu.sync_copy(dst_hbm.at[pl.ds(base, SB)], dstv)
        pltpu.sync_copy(et_hbm.at[pl.ds(base, SB)], etv)

        def idx_body(g, carry2):
            o = g * G
            for k in range(G // 16):
                dv = dstv[pl.ds(o + 16 * k, 16)]
                ev = etv[pl.ds(o + 16 * k, 16)]
                segb[g, pl.ds(16 * k, 16)] = dv * N_REL + ev
            return carry2
        lax.fori_loop(0, NG, idx_body, 0)

        def scat_body(g, carry2):
            pltpu.sync_copy(ones, cnt_s.at[segb.at[g]], add=True)
            return carry2
        lax.fori_loop(0, NG, scat_body, 0)
        return carry
    lax.fori_loop(0, NSUPER_CNT, super_body, 0)
    plsc.subcore_barrier()

    pltpu.sync_copy(cnt_s.at[pl.ds(acc_base, ROWS_PER_TILE)],
                    cnt_out.at[c, pl.ds(acc_base, ROWS_PER_TILE)])


def _make_sc_counts():
    mesh = plsc.VectorSubcoreMesh(core_axis_name="c", subcore_axis_name="s")
    scratch = (
        pltpu.VMEM_SHARED((SEG, 16), jnp.float32),
        pltpu.VMEM((SB,), jnp.int32),
        pltpu.VMEM((SB,), jnp.int32),
        pltpu.VMEM((NG, G), jnp.int32),
        pltpu.VMEM((G, 16), jnp.float32),
        pltpu.VMEM((ZR, 16), jnp.float32),
    )
    return pl.kernel(
        _cnt_body, mesh=mesh,
        out_type=(jax.ShapeDtypeStruct((2, SEG, 16), jnp.float32),),
        scratch_types=scratch,
        compiler_params=pltpu.CompilerParams(use_tc_tiling_on_sc=False))


def _dense_body(final_avg, x_ref, acc_ref, inv_ref, root_ref, w_ref, b_ref,
                x0_ref, out_ref):
    x = x_ref[...]
    out = jnp.dot(x, root_ref[...], preferred_element_type=jnp.float32)
    out += b_ref[...]
    inv = inv_ref[...]
    scaled = [acc_ref[ch] * inv for ch in range(NCHUNK)]
    pieces = [scaled[ch][:, r * CW:(r + 1) * CW]
              for r in range(N_REL) for ch in range(NCHUNK)]
    bcat = jnp.concatenate(pieces, axis=1)
    out += jnp.dot(bcat, w_ref[...], preferred_element_type=jnp.float32)
    if final_avg:
        out_ref[...] = (x0_ref[...] + x + out) * (1.0 / 3.0)
    else:
        out_ref[...] = out


def _dense_layer(x, acc_t, inv_t, root, wstack, b, x0, final_avg):
    grid = (N_NODES // BN,)
    return pl.pallas_call(
        functools.partial(_dense_body, final_avg),
        grid=grid,
        in_specs=[
            pl.BlockSpec((BN, EMB), lambda i: (i, 0)),
            pl.BlockSpec((NCHUNK, BN, EMB), lambda i: (0, i, 0)),
            pl.BlockSpec((BN, EMB), lambda i: (i, 0)),
            pl.BlockSpec((EMB, EMB), lambda i: (0, 0)),
            pl.BlockSpec((N_REL * EMB, EMB), lambda i: (0, 0)),
            pl.BlockSpec((1, EMB), lambda i: (0, 0)),
            pl.BlockSpec((BN, EMB), lambda i: (i, 0)),
        ],
        out_specs=pl.BlockSpec((BN, EMB), lambda i: (i, 0)),
        out_shape=jax.ShapeDtypeStruct((N_NODES, EMB), jnp.float32),
    )(x, acc_t, inv_t, root, wstack, b, x0)


_sc_segsum = _make_sc_segsum()
_sc_counts = _make_sc_counts()


def kernel(edge_index_mp, edge_type, emb, w0, root0, b0, w1, root1, b1):
    src = edge_index_mp[0]
    dst = edge_index_mp[1]

    (cnt16,) = _sc_counts(dst, edge_type)
    cnts = cnt16[0, :, 0] + cnt16[1, :, 0]
    inv = 1.0 / jnp.clip(cnts.reshape(N_NODES, N_REL), 1.0, None)
    inv_t = jnp.repeat(inv, CW, axis=1)

    (acc0,) = _sc_segsum(src, dst, edge_type, emb.reshape(SEG, CW))
    x1 = _dense_layer(emb, acc0.reshape(NCHUNK, N_NODES, EMB), inv_t,
                      root0, w0.reshape(N_REL * EMB, EMB),
                      b0.reshape(1, EMB), emb, False)
    (acc1,) = _sc_segsum(src, dst, edge_type, x1.reshape(SEG, CW))
    x2f = _dense_layer(x1, acc1.reshape(NCHUNK, N_NODES, EMB), inv_t,
                       root1, w1.reshape(N_REL * EMB, EMB),
                       b1.reshape(1, EMB), emb, True)
    return x2f
